# baseline (speedup 1.0000x reference)
import functools

import jax
import jax.numpy as jnp
from jax import lax
from jax.experimental import pallas as pl
from jax.experimental.pallas import tpu as pltpu
from jax.experimental.pallas import tpu_sc as plsc

NUM_CORES = 2
NUM_SUBCORES = 16
LANES = 16
NW = NUM_CORES * NUM_SUBCORES
B = 16384
D = 64
BPW = B // NW
CHUNK = 128
NCH = BPW // CHUNK
KD = D // LANES


def _rsqrt(x):
  i = lax.bitcast_convert_type(x, jnp.int32)
  i = jnp.int32(0x5F3759DF) - lax.shift_right_logical(i, 1)
  y = lax.bitcast_convert_type(i, jnp.float32)
  for _ in range(2):
    y = y * (1.5 - 0.5 * x * y * y)
  return y


def _body(hidx_hbm, ridx_hbm, tidx_hbm, ent_hbm, rel_hbm, out_hbm,
          hi_v, ri_v, ti_v, h_v, r_v, t_v, out_v, sem):
  wid = lax.axis_index("s") * NUM_CORES + lax.axis_index("c")

  pltpu.sync_copy(hidx_hbm.at[wid], hi_v)
  pltpu.sync_copy(ridx_hbm.at[wid], ri_v)
  pltpu.sync_copy(tidx_hbm.at[wid], ti_v)

  copies = []
  for j in range(NCH):
    dst = pl.ds(j * CHUNK, CHUNK)
    copies.append(pltpu.async_copy(ent_hbm.at[hi_v.at[j]], h_v.at[dst], sem))
    copies.append(pltpu.async_copy(rel_hbm.at[ri_v.at[j]], r_v.at[dst], sem))
    copies.append(pltpu.async_copy(ent_hbm.at[ti_v.at[j]], t_v.at[dst], sem))
  for c in copies:
    c.wait()

  lanes_iota = lax.iota(jnp.int32, LANES)

  def group(g, carry):
    def item(k, svec):
      i = g * LANES + k

      def row(ref):
        parts = []
        acc = None
        for kk in range(KD):
          v = ref[i, pl.ds(kk * LANES, LANES)]
          parts.append(v)
          acc = v * v if acc is None else acc + v * v
        return parts, jnp.sum(acc)

      hp, sh = row(h_v)
      rp, sr = row(r_v)
      tp, st = row(t_v)
      ih = _rsqrt(jnp.maximum(sh, 1e-24))
      ir = _rsqrt(jnp.maximum(sr, 1e-24))
      it = _rsqrt(jnp.maximum(st, 1e-24))
      acc = None
      for kk in range(KD):
        term = jnp.abs(hp[kk] * ih + rp[kk] * ir - tp[kk] * it)
        acc = term if acc is None else acc + term
      s = jnp.sum(acc)
      return jnp.where(lanes_iota == k, s, svec)

    svec = lax.fori_loop(0, LANES, item, jnp.zeros((LANES,), jnp.float32),
                         unroll=4)
    out_v[pl.ds(g * LANES, LANES)] = svec
    return carry

  lax.fori_loop(0, BPW // LANES, group, 0)

  pltpu.sync_copy(out_v, out_hbm.at[pl.ds(wid * BPW, BPW)])


@functools.partial(
    pl.kernel,
    out_type=jax.ShapeDtypeStruct((B,), jnp.float32),
    mesh=plsc.VectorSubcoreMesh(core_axis_name="c", subcore_axis_name="s"),
    compiler_params=pltpu.CompilerParams(
        needs_layout_passes=False, use_tc_tiling_on_sc=False),
    scratch_types=[
        pltpu.VMEM((NCH, CHUNK), jnp.int32),
        pltpu.VMEM((NCH, CHUNK), jnp.int32),
        pltpu.VMEM((NCH, CHUNK), jnp.int32),
        pltpu.VMEM((BPW, D), jnp.float32),
        pltpu.VMEM((BPW, D), jnp.float32),
        pltpu.VMEM((BPW, D), jnp.float32),
        pltpu.VMEM((BPW,), jnp.float32),
        pltpu.SemaphoreType.DMA,
    ],
)
def _transe_sc(hidx, ridx, tidx, ent_emb, rel_emb, out,
               hi_v, ri_v, ti_v, h_v, r_v, t_v, out_v, sem):
  _body(hidx, ridx, tidx, ent_emb, rel_emb, out,
        hi_v, ri_v, ti_v, h_v, r_v, t_v, out_v, sem)


def kernel(data, ent_emb, rel_emb):
  data = data.astype(jnp.int32)
  hidx = data[:, 0].reshape(NW, NCH, CHUNK)
  ridx = data[:, 1].reshape(NW, NCH, CHUNK)
  tidx = data[:, 2].reshape(NW, NCH, CHUNK)
  return _transe_sc(hidx, ridx, tidx, ent_emb, rel_emb)


# speedup vs baseline: 1.1005x; 1.1005x over previous
import functools

import jax
import jax.numpy as jnp
from jax import lax
from jax.experimental import pallas as pl
from jax.experimental.pallas import tpu as pltpu
from jax.experimental.pallas import tpu_sc as plsc

NUM_CORES = 2
NUM_SUBCORES = 16
LANES = 16
NW = NUM_CORES * NUM_SUBCORES
B = 16384
D = 64
DPAD = 128
BPW = B // NW
CHUNK = 128
NCH = BPW // CHUNK
KD = D // LANES


def _rsqrt(x):
  i = lax.bitcast_convert_type(x, jnp.int32)
  i = jnp.int32(0x5F3759DF) - lax.shift_right_logical(i, 1)
  y = lax.bitcast_convert_type(i, jnp.float32)
  for _ in range(2):
    y = y * (1.5 - 0.5 * x * y * y)
  return y


def _body(hidx_hbm, ridx_hbm, tidx_hbm, ent_hbm, rel_hbm, out_hbm,
          hi_v, ri_v, ti_v, h_v, r_v, t_v, out_v, sem):
  wid = lax.axis_index("s") * NUM_CORES + lax.axis_index("c")

  pltpu.sync_copy(hidx_hbm.at[wid], hi_v)
  pltpu.sync_copy(ridx_hbm.at[wid], ri_v)
  pltpu.sync_copy(tidx_hbm.at[wid], ti_v)

  lanes_iota = lax.iota(jnp.int32, LANES)

  def chunk(j, carry):
    ch = pltpu.async_copy(ent_hbm.at[hi_v.at[j]], h_v, sem)
    cr = pltpu.async_copy(rel_hbm.at[ri_v.at[j]], r_v, sem)
    ct = pltpu.async_copy(ent_hbm.at[ti_v.at[j]], t_v, sem)
    ch.wait()
    cr.wait()
    ct.wait()

    def group(g, carry2):
      def item(k, svec):
        i = g * LANES + k

        def row(ref):
          parts = []
          acc = None
          for kk in range(KD):
            v = ref[i, pl.ds(kk * LANES, LANES)]
            parts.append(v)
            acc = v * v if acc is None else acc + v * v
          return parts, jnp.sum(acc)

        hp, sh = row(h_v)
        rp, sr = row(r_v)
        tp, st = row(t_v)
        ih = _rsqrt(jnp.maximum(sh, 1e-24))
        ir = _rsqrt(jnp.maximum(sr, 1e-24))
        it = _rsqrt(jnp.maximum(st, 1e-24))
        acc = None
        for kk in range(KD):
          term = jnp.abs(hp[kk] * ih + rp[kk] * ir - tp[kk] * it)
          acc = term if acc is None else acc + term
        s = jnp.sum(acc)
        return jnp.where(lanes_iota == k, s, svec)

      svec = lax.fori_loop(0, LANES, item, jnp.zeros((LANES,), jnp.float32),
                           unroll=4)
      out_v[pl.ds(j * CHUNK + g * LANES, LANES)] = svec
      return carry2

    lax.fori_loop(0, CHUNK // LANES, group, 0)
    return carry

  lax.fori_loop(0, NCH, chunk, 0)

  pltpu.sync_copy(out_v, out_hbm.at[pl.ds(wid * BPW, BPW)])


@functools.partial(
    pl.kernel,
    out_type=jax.ShapeDtypeStruct((B,), jnp.float32),
    mesh=plsc.VectorSubcoreMesh(core_axis_name="c", subcore_axis_name="s"),
    compiler_params=pltpu.CompilerParams(
        needs_layout_passes=False, use_tc_tiling_on_sc=True),
    scratch_types=[
        pltpu.VMEM((NCH, CHUNK), jnp.int32),
        pltpu.VMEM((NCH, CHUNK), jnp.int32),
        pltpu.VMEM((NCH, CHUNK), jnp.int32),
        pltpu.VMEM((CHUNK, DPAD), jnp.float32),
        pltpu.VMEM((CHUNK, DPAD), jnp.float32),
        pltpu.VMEM((CHUNK, DPAD), jnp.float32),
        pltpu.VMEM((BPW,), jnp.float32),
        pltpu.SemaphoreType.DMA,
    ],
)
def _transe_sc(hidx, ridx, tidx, ent_emb, rel_emb, out,
               hi_v, ri_v, ti_v, h_v, r_v, t_v, out_v, sem):
  _body(hidx, ridx, tidx, ent_emb, rel_emb, out,
        hi_v, ri_v, ti_v, h_v, r_v, t_v, out_v, sem)


def kernel(data, ent_emb, rel_emb):
  data = data.astype(jnp.int32)
  hidx = data[:, 0].reshape(NW, NCH, CHUNK)
  ridx = data[:, 1].reshape(NW, NCH, CHUNK)
  tidx = data[:, 2].reshape(NW, NCH, CHUNK)
  ent_pad = jnp.pad(ent_emb, ((0, 0), (0, DPAD - D)))
  rel_pad = jnp.pad(rel_emb, ((0, 0), (0, DPAD - D)))
  return _transe_sc(hidx, ridx, tidx, ent_pad, rel_pad)
